# baseline (speedup 1.0000x reference)
import jax
import jax.numpy as jnp
from jax import lax
from jax.experimental import pallas as pl
from jax.experimental.pallas import tpu as pltpu
from jax.experimental.pallas import tpu_sc as plsc

B, S, D = 4, 2048, 1024
NC, NS = 2, 16
NW = NC * NS
SEG = S // NW
SUB = 16
QPS = SEG // SUB
NT = B * QPS
GPR = D // 16
NBUF = 3


def _body(x_ref, tok_ref, pos_ref, out_ref, idx_v, pbuf, buf0, buf1, buf2,
          psem, gs0, gs1, gs2, ws0, ws1, ws2):
    c = lax.axis_index("c")
    s = lax.axis_index("s")
    wid = s * NC + c
    bufs = (buf0, buf1, buf2)
    gsems = (gs0, gs1, gs2)
    wsems = (ws0, ws1, ws2)

    pltpu.sync_copy(x_ref.at[wid], idx_v)

    def start_gather(t):
        return pltpu.async_copy(tok_ref.at[idx_v.at[t]], bufs[t % NBUF],
                                gsems[t % NBUF])

    gd = [start_gather(0), start_gather(1), None]
    pd = pltpu.async_copy(pos_ref.at[pl.ds(wid * SEG, SEG)], pbuf, psem)
    wb = [None, None, None]
    for t in range(NT):
        p = t % NBUF
        if t + 2 < NT:
            np_ = (t + 2) % NBUF
            if wb[np_] is not None:
                wb[np_].wait()
                wb[np_] = None
            gd[np_] = start_gather(t + 2)
        gd[p].wait()
        if t == 0:
            pd.wait()

        q = t % QPS
        cur = bufs[p]

        @plsc.parallel_loop(0, SUB * GPR, unroll=32)
        def _add(i):
            r = i // GPR
            k = (i % GPR) * 16
            plsc.addupdate(cur.at[r, pl.ds(k, 16)],
                           pbuf[q * SUB + r, pl.ds(k, 16)])

        b = t // QPS
        base = b * S + wid * SEG + q * SUB
        wb[p] = pltpu.async_copy(cur, out_ref.at[pl.ds(base, SUB)], wsems[p])
    for d in wb:
        if d is not None:
            d.wait()


@jax.jit
def _emb(xr, token_table, pos_table):
    kern = pl.kernel(
        _body,
        out_type=jax.ShapeDtypeStruct((B * S, D), jnp.float32),
        mesh=plsc.VectorSubcoreMesh(core_axis_name="c", subcore_axis_name="s"),
        scratch_types=[
            pltpu.VMEM((NT, SUB), jnp.int32),
            pltpu.VMEM((SEG, D), jnp.float32),
            pltpu.VMEM((SUB, D), jnp.float32),
            pltpu.VMEM((SUB, D), jnp.float32),
            pltpu.VMEM((SUB, D), jnp.float32),
            pltpu.SemaphoreType.DMA,
            pltpu.SemaphoreType.DMA,
            pltpu.SemaphoreType.DMA,
            pltpu.SemaphoreType.DMA,
            pltpu.SemaphoreType.DMA,
            pltpu.SemaphoreType.DMA,
            pltpu.SemaphoreType.DMA,
        ],
    )
    return kern(xr, token_table, pos_table)


def kernel(x, token_table, pos_table):
    xr = (x.astype(jnp.int32)
          .reshape(B, NW, NT // B, SUB)
          .transpose(1, 0, 2, 3)
          .reshape(NW, NT, SUB))
    out = _emb(xr, token_table, pos_table)
    return out.reshape(B, S, D)


# speedup vs baseline: 1.0583x; 1.0583x over previous
import jax
import jax.numpy as jnp
from jax import lax
from jax.experimental import pallas as pl
from jax.experimental.pallas import tpu as pltpu
from jax.experimental.pallas import tpu_sc as plsc

B, S, D = 4, 2048, 1024
NC, NS = 2, 16
NW = NC * NS
SEG = S // NW
SUB = 16
QPS = SEG // SUB
NT = B * QPS
GPR = D // 16


def _body(x_ref, tok_ref, pos_ref, out_ref, idx_v, pbuf, buf0, buf1,
          psem, gs0, gs1, ws0, ws1):
    c = lax.axis_index("c")
    s = lax.axis_index("s")
    wid = s * NC + c
    bufs = (buf0, buf1)
    gsems = (gs0, gs1)
    wsems = (ws0, ws1)

    pltpu.sync_copy(x_ref.at[wid], idx_v)

    def start_gather(t):
        return pltpu.async_copy(tok_ref.at[idx_v.at[t]], bufs[t % 2],
                                gsems[t % 2])

    gd = [start_gather(0), None]
    pd = pltpu.async_copy(pos_ref.at[pl.ds(wid * SEG, SEG)], pbuf, psem)
    wb = [None, None]
    for t in range(NT):
        p = t % 2
        if t + 1 < NT:
            if wb[1 - p] is not None:
                wb[1 - p].wait()
                wb[1 - p] = None
            gd[1 - p] = start_gather(t + 1)
        gd[p].wait()
        if t == 0:
            pd.wait()

        q = t % QPS
        cur = bufs[p]

        @plsc.parallel_loop(0, SUB * GPR, unroll=8)
        def _add(i):
            r = i // GPR
            k = (i % GPR) * 16
            plsc.addupdate(cur.at[r, pl.ds(k, 16)],
                           pbuf[q * SUB + r, pl.ds(k, 16)])

        b = t // QPS
        base = b * S + wid * SEG + q * SUB
        wb[p] = pltpu.async_copy(cur, out_ref.at[pl.ds(base, SUB)], wsems[p])
    for d in wb:
        if d is not None:
            d.wait()


@jax.jit
def _emb(xr, token_table, pos_table):
    kern = pl.kernel(
        _body,
        out_type=jax.ShapeDtypeStruct((B * S, D), jnp.float32),
        mesh=plsc.VectorSubcoreMesh(core_axis_name="c", subcore_axis_name="s"),
        scratch_types=[
            pltpu.VMEM((NT, SUB), jnp.int32),
            pltpu.VMEM((SEG, D), jnp.float32),
            pltpu.VMEM((SUB, D), jnp.float32),
            pltpu.VMEM((SUB, D), jnp.float32),
            pltpu.SemaphoreType.DMA,
            pltpu.SemaphoreType.DMA,
            pltpu.SemaphoreType.DMA,
            pltpu.SemaphoreType.DMA,
            pltpu.SemaphoreType.DMA,
        ],
    )
    return kern(xr, token_table, pos_table)


def kernel(x, token_table, pos_table):
    xr = (x.astype(jnp.int32)
          .reshape(B, NW, NT // B, SUB)
          .transpose(1, 0, 2, 3)
          .reshape(NW, NT, SUB))
    out = _emb(xr, token_table, pos_table)
    return out.reshape(B, S, D)
